# baseline (speedup 1.0000x reference)
import functools

import jax
import jax.numpy as jnp
import numpy as np
from jax import lax
from jax.experimental import pallas as pl
from jax.experimental.pallas import tpu as pltpu
from jax.experimental.pallas import tpu_sc as plsc

EMBED_DIM = 32
SCALE = float(np.sqrt(np.float32(EMBED_DIM)))

B_TOTAL = 16384 * 50
NUM_WORKERS = 32
B_PER_W = B_TOTAL // NUM_WORKERS
CHUNK = 128
N_CHUNKS = B_PER_W // CHUNK


def _embed_body(x_hbm, table_hbm, out_hbm, idx_v, rows_v, gsem):
    wid = lax.axis_index("s") * 2 + lax.axis_index("c")
    base = wid * B_PER_W

    pltpu.sync_copy(x_hbm.at[pl.ds(wid * N_CHUNKS, N_CHUNKS)], idx_v)

    def step(c, carry):
        off = base + c * CHUNK
        pltpu.async_copy(table_hbm.at[idx_v.at[c]], rows_v, gsem).wait()

        def scale_row(r, carry2):
            v0 = rows_v[r, pl.ds(0, 16)]
            rows_v[r, pl.ds(0, 16)] = v0 * SCALE
            v1 = rows_v[r, pl.ds(16, 16)]
            rows_v[r, pl.ds(16, 16)] = v1 * SCALE
            return carry2

        lax.fori_loop(0, CHUNK, scale_row, 0, unroll=4)
        pltpu.sync_copy(rows_v, out_hbm.at[pl.ds(off, CHUNK)])
        return carry

    lax.fori_loop(0, N_CHUNKS, step, 0)


@jax.jit
def _embed(x2d, table):
    mesh = plsc.VectorSubcoreMesh(core_axis_name="c", subcore_axis_name="s")
    f = pl.kernel(
        _embed_body,
        mesh=mesh,
        out_type=jax.ShapeDtypeStruct((B_TOTAL, EMBED_DIM), jnp.float32),
        scratch_types=[
            pltpu.VMEM((N_CHUNKS, CHUNK), jnp.int32),
            pltpu.VMEM((CHUNK, EMBED_DIM), jnp.float32),
            pltpu.SemaphoreType.DMA,
        ],
        compiler_params=pltpu.CompilerParams(use_tc_tiling_on_sc=False),
    )
    return f(x2d, table)


def kernel(x, table):
    x2d = x.reshape(NUM_WORKERS * N_CHUNKS, CHUNK).astype(jnp.int32)
    out = _embed(x2d, table)
    return out.reshape(x.shape[0], x.shape[1], EMBED_DIM)


# speedup vs baseline: 1.0056x; 1.0056x over previous
import jax
import jax.numpy as jnp
import numpy as np
from jax import lax
from jax.experimental import pallas as pl
from jax.experimental.pallas import tpu as pltpu
from jax.experimental.pallas import tpu_sc as plsc

EMBED_DIM = 32
SCALE = float(np.sqrt(np.float32(EMBED_DIM)))

B_TOTAL = 16384 * 50
NUM_WORKERS = 32
B_PER_W = B_TOTAL // NUM_WORKERS
CHUNK = 128
N_CHUNKS = B_PER_W // CHUNK
NBUF = 4


def _embed_body(x_hbm, table_hbm, out_hbm, idx_v, g_v, o_v, *sems):
    gsems = sems[:NBUF]
    osems = sems[NBUF:]
    wid = lax.axis_index("s") * 2 + lax.axis_index("c")
    base = wid * B_PER_W

    pltpu.sync_copy(x_hbm.at[pl.ds(wid * N_CHUNKS, N_CHUNKS)], idx_v)

    def fire_gather(c, b):
        pltpu.async_copy(table_hbm.at[idx_v.at[c]], g_v.at[b], gsems[b])

    def scale_chunk(b):
        def scale_row(r, carry):
            o_v[b, r, pl.ds(0, 16)] = g_v[b, r, pl.ds(0, 16)] * SCALE
            o_v[b, r, pl.ds(16, 16)] = g_v[b, r, pl.ds(16, 16)] * SCALE
            return carry

        lax.fori_loop(0, CHUNK, scale_row, 0, unroll=8)

    for b in range(NBUF):
        fire_gather(b, b)

    @pl.loop(0, N_CHUNKS, step=NBUF)
    def step(c0):
        for b in range(NBUF):
            c = c0 + b
            off = base + c * CHUNK
            pltpu.make_async_copy(
                table_hbm.at[idx_v.at[c]], g_v.at[b], gsems[b]).wait()

            @pl.when(c >= NBUF)
            def _():
                off_prev = base + (c - NBUF) * CHUNK
                pltpu.make_async_copy(
                    o_v.at[b], out_hbm.at[pl.ds(off_prev, CHUNK)],
                    osems[b]).wait()

            scale_chunk(b)
            pltpu.async_copy(
                o_v.at[b], out_hbm.at[pl.ds(off, CHUNK)], osems[b])

            @pl.when(c + NBUF < N_CHUNKS)
            def _():
                fire_gather(c + NBUF, b)

    for b in range(NBUF):
        c = N_CHUNKS - NBUF + b
        off = base + c * CHUNK
        pltpu.make_async_copy(
            o_v.at[b], out_hbm.at[pl.ds(off, CHUNK)], osems[b]).wait()


@jax.jit
def _embed(x2d, table):
    mesh = plsc.VectorSubcoreMesh(core_axis_name="c", subcore_axis_name="s")
    f = pl.kernel(
        _embed_body,
        mesh=mesh,
        out_type=jax.ShapeDtypeStruct((B_TOTAL, EMBED_DIM), jnp.float32),
        scratch_types=[
            pltpu.VMEM((N_CHUNKS, CHUNK), jnp.int32),
            pltpu.VMEM((NBUF, CHUNK, EMBED_DIM), jnp.float32),
            pltpu.VMEM((NBUF, CHUNK, EMBED_DIM), jnp.float32),
        ] + [pltpu.SemaphoreType.DMA] * (2 * NBUF),
        compiler_params=pltpu.CompilerParams(use_tc_tiling_on_sc=False),
    )
    return f(x2d, table)


def kernel(x, table):
    x2d = x.reshape(NUM_WORKERS * N_CHUNKS, CHUNK).astype(jnp.int32)
    out = _embed(x2d, table)
    return out.reshape(x.shape[0], x.shape[1], EMBED_DIM)


# speedup vs baseline: 1.6862x; 1.6768x over previous
import jax
import jax.numpy as jnp
import numpy as np
from jax import lax
from jax.experimental import pallas as pl
from jax.experimental.pallas import tpu as pltpu
from jax.experimental.pallas import tpu_sc as plsc

EMBED_DIM = 32
SCALE = float(np.sqrt(np.float32(EMBED_DIM)))

NROWS = 16384
SEQ = 50
NUM_WORKERS = 32
ROWS_PER_W = NROWS // NUM_WORKERS
ROWS_PER_CHUNK = 2
CHUNK = ROWS_PER_CHUNK * SEQ
N_CHUNKS = ROWS_PER_W // ROWS_PER_CHUNK
NBUF = 4


def _embed_body(x_hbm, table_hbm, out_hbm, idx_v, g_v, o_v, *sems):
    gsems = sems[:NBUF]
    osems = sems[NBUF:]
    wid = lax.axis_index("s") * 2 + lax.axis_index("c")
    row0 = wid * ROWS_PER_W

    pltpu.sync_copy(x_hbm.at[pl.ds(wid * N_CHUNKS, N_CHUNKS)], idx_v)

    def fire_gather(c, b):
        pltpu.async_copy(table_hbm.at[idx_v.at[c]], g_v.at[b], gsems[b])

    def wait_gather(c, b):
        pltpu.make_async_copy(
            table_hbm.at[idx_v.at[c]], g_v.at[b], gsems[b]).wait()

    def fire_out(c, b):
        pltpu.async_copy(
            o_v.at[b],
            out_hbm.at[pl.ds(row0 + c * ROWS_PER_CHUNK, ROWS_PER_CHUNK)],
            osems[b])

    def wait_out(c, b):
        pltpu.make_async_copy(
            o_v.at[b],
            out_hbm.at[pl.ds(row0 + c * ROWS_PER_CHUNK, ROWS_PER_CHUNK)],
            osems[b]).wait()

    def scale_chunk(b):
        for r in range(ROWS_PER_CHUNK):
            def scale_tok(s, carry):
                g0 = g_v[b, r * SEQ + s, pl.ds(0, 16)]
                o_v[b, r, s, pl.ds(0, 16)] = g0 * SCALE
                g1 = g_v[b, r * SEQ + s, pl.ds(16, 16)]
                o_v[b, r, s, pl.ds(16, 16)] = g1 * SCALE
                return carry

            lax.fori_loop(0, SEQ, scale_tok, 0, unroll=5)

    for b in range(NBUF):
        fire_gather(b, b)

    @pl.loop(0, N_CHUNKS, step=NBUF)
    def step(c0):
        for b in range(NBUF):
            c = c0 + b
            wait_gather(c, b)

            @pl.when(c >= NBUF)
            def _():
                wait_out(c - NBUF, b)

            scale_chunk(b)
            fire_out(c, b)

            @pl.when(c + NBUF < N_CHUNKS)
            def _():
                fire_gather(c + NBUF, b)

    for b in range(NBUF):
        wait_out(N_CHUNKS - NBUF + b, b)


@jax.jit
def _embed(x2d, table):
    mesh = plsc.VectorSubcoreMesh(core_axis_name="c", subcore_axis_name="s")
    f = pl.kernel(
        _embed_body,
        mesh=mesh,
        out_type=jax.ShapeDtypeStruct((NROWS, SEQ, EMBED_DIM), jnp.float32),
        scratch_types=[
            pltpu.VMEM((N_CHUNKS, CHUNK), jnp.int32),
            pltpu.VMEM((NBUF, CHUNK, EMBED_DIM), jnp.float32),
            pltpu.VMEM((NBUF, ROWS_PER_CHUNK, SEQ, EMBED_DIM), jnp.float32),
        ] + [pltpu.SemaphoreType.DMA] * (2 * NBUF),
        compiler_params=pltpu.CompilerParams(use_tc_tiling_on_sc=False),
    )
    return f(x2d, table)


def kernel(x, table):
    x2d = x.reshape(NUM_WORKERS * N_CHUNKS, CHUNK).astype(jnp.int32)
    return _embed(x2d, table)
